# baseline (speedup 1.0000x reference)
import jax
import jax.numpy as jnp
from jax.experimental import pallas as pl
from jax.experimental.pallas import tpu as pltpu

_BM = 1024
_NBUF = 4


def _router_body(x_hbm, wt_ref, idx_ref, pw_ref, xbuf, sems):
    m_tot = x_hbm.shape[0]
    e_dim = wt_ref.shape[1]
    nsteps = m_tot // _BM
    wt = wt_ref[...]
    ones = jnp.ones((e_dim, e_dim), dtype=jnp.float32)
    lane = jax.lax.broadcasted_iota(jnp.int32, (1, e_dim), 1)
    w2 = jax.lax.bitcast_convert_type((127 - lane) << 23, jnp.float32)

    def slab_copy(step, slot):
        return pltpu.make_async_copy(
            x_hbm.at[pl.ds(step * _BM, _BM), :], xbuf.at[slot], sems.at[slot])

    for b in range(min(_NBUF - 1, nsteps)):
        slab_copy(b, b).start()

    def step_fn(i, carry):
        nxt = i + _NBUF - 1

        @pl.when(nxt < nsteps)
        def _():
            slab_copy(nxt, nxt % _NBUF).start()

        slot = jax.lax.rem(i, _NBUF)
        slab_copy(i, slot).wait()
        xs = xbuf[slot]
        logits = jnp.dot(xs, wt, preferred_element_type=jnp.float32)
        m = jnp.max(logits, axis=-1, keepdims=True)
        e = jnp.exp(logits - m)
        s = jax.lax.dot_general(e, ones, (((1,), (0,)), ((), ())),
                                preferred_element_type=jnp.float32)
        pw_ref[pl.ds(i * _BM, _BM), :] = e * (1.0 / s)
        v = jnp.where(logits == m, w2, 0.0)
        t = jax.lax.dot_general(v, ones, (((1,), (0,)), ((), ())),
                                preferred_element_type=jnp.float32)
        bits = jax.lax.bitcast_convert_type(t[:, :1], jnp.int32)
        idx = jnp.maximum(127 - (bits >> 23), 0)
        rows = _BM // 128
        idx_ref[pl.ds(i * rows, rows), :] = idx.reshape((rows, 128))
        return carry

    jax.lax.fori_loop(0, nsteps, step_fn, 0)


def kernel(x, W):
    M, K = x.shape
    E = W.shape[0]
    wt = W.T
    idx, pw = pl.pallas_call(
        _router_body,
        in_specs=[
            pl.BlockSpec(memory_space=pltpu.HBM),
            pl.BlockSpec(memory_space=pltpu.VMEM),
        ],
        out_specs=[
            pl.BlockSpec(memory_space=pltpu.VMEM),
            pl.BlockSpec(memory_space=pltpu.VMEM),
        ],
        out_shape=[
            jax.ShapeDtypeStruct((M // 128, 128), jnp.int32),
            jax.ShapeDtypeStruct((M, E), jnp.float32),
        ],
        scratch_shapes=[
            pltpu.VMEM((_NBUF, _BM, K), jnp.float32),
            pltpu.SemaphoreType.DMA((_NBUF,)),
        ],
    )(x, wt)
    return idx.reshape((M,)), pw
